# baseline (speedup 1.0000x reference)
import functools

import jax
import jax.numpy as jnp
from jax import lax
from jax.experimental import pallas as pl
from jax.experimental.pallas import tpu as pltpu
from jax.experimental.pallas import tpu_sc as plsc

_SEG = 2048
_HW = 196
_N_SC_SEG = 8
_QUARTERS = 4
_SND_CHUNK = 32
_IMG_ROWS = 392


def _make_sc_kernel(C, B):
    info = plsc.get_sparse_core_info()
    nw = info.num_cores * info.num_subcores
    snd_rows_w = _SEG // _QUARTERS
    n_snd_chunks = snd_rows_w // _SND_CHUNK
    ng = C // 16
    planes_w = B * C // nw
    p_chunk = _IMG_ROWS * 128 // _HW
    n_img_chunks = planes_w // p_chunk
    img_out_rows = planes_w // C
    mesh = plsc.VectorSubcoreMesh(core_axis_name="c", subcore_axis_name="s")

    @functools.partial(
        pl.kernel,
        out_type=(
            jax.ShapeDtypeStruct((_N_SC_SEG, _QUARTERS, C), jnp.float32),
            jax.ShapeDtypeStruct((B, C), jnp.float32),
        ),
        mesh=mesh,
        scratch_types=[
            pltpu.VMEM((2, _SND_CHUNK, C), jnp.float32),
            pltpu.VMEM((2, _IMG_ROWS, 128), jnp.float32),
            pltpu.VMEM((C,), jnp.float32),
            pltpu.VMEM((img_out_rows, C), jnp.float32),
            pltpu.SemaphoreType.DMA,
            pltpu.SemaphoreType.DMA,
            pltpu.SemaphoreType.DMA,
            pltpu.SemaphoreType.DMA,
        ],
        compiler_params=pltpu.CompilerParams(needs_layout_passes=False),
    )
    def sc_body(z_hbm, zi_hbm, osnd_hbm, oimg_hbm,
                sbuf, ibuf, row_v, img_v, s0, s1, s2, s3):
        wid = lax.axis_index("s") * info.num_cores + lax.axis_index("c")

        rbase = wid * (planes_w * _HW // 128)
        isems = (s2, s3)

        def icopy(k):
            return pltpu.make_async_copy(
                zi_hbm.at[pl.ds(rbase + k * _IMG_ROWS, _IMG_ROWS), :],
                ibuf.at[k % 2], isems[k % 2])

        icopy(0).start()

        base = wid * snd_rows_w
        ssems = (s0, s1)

        def scopy(k):
            return pltpu.make_async_copy(
                z_hbm.at[pl.ds(base + k * _SND_CHUNK, _SND_CHUNK), :],
                sbuf.at[k % 2], ssems[k % 2])

        scopy(0).start()
        accs = tuple(jnp.zeros((16,), jnp.float32) for _ in range(ng))
        for k in range(n_snd_chunks):
            if k + 1 < n_snd_chunks:
                scopy(k + 1).start()
            scopy(k).wait()
            slot = sbuf.at[k % 2]

            def sbody(r, a, slot=slot):
                return tuple(
                    a[c] + slot[r, c * 16:(c + 1) * 16] for c in range(ng))

            accs = lax.fori_loop(0, _SND_CHUNK, sbody, accs)
        for c in range(ng):
            row_v[c * 16:(c + 1) * 16] = accs[c]
        pltpu.sync_copy(row_v, osnd_hbm.at[wid // _QUARTERS, wid % _QUARTERS])

        lane = lax.iota(jnp.int32, 16)
        for k in range(n_img_chunks):
            if k + 1 < n_img_chunks:
                icopy(k + 1).start()
            icopy(k).wait()
            slot = ibuf.at[k % 2]
            for g in range(p_chunk // 16):
                idx0 = (g * 16 + lane) * _HW

                def gbody(i, a, slot=slot, idx0=idx0):
                    t0 = i * 7
                    for u in range(7):
                        f = idx0 + (t0 + u)
                        a = a + plsc.load_gather(
                            slot, [lax.shift_right_logical(f, 7),
                                   lax.bitwise_and(f, 127)])
                    return a

                acc = lax.fori_loop(0, _HW // 7, gbody,
                                    jnp.zeros((16,), jnp.float32))
                flat = k * p_chunk + g * 16
                img_v[flat // C, flat % C:flat % C + 16] = acc
        pltpu.sync_copy(
            img_v, oimg_hbm.at[pl.ds(wid * img_out_rows, img_out_rows), :])

    return sc_body


def _tc_snd_body(x_ref, o_ref):
    o_ref[...] = jnp.sum(x_ref[...], axis=1, keepdims=True)


def _bcast_body(inv_ref, img_ref, sc_ref, tc_ref, mimg_ref, msnd_ref):
    i = pl.program_id(0)
    mimg_ref[...] = (img_ref[...] * (1.0 / _HW))[None, :, :]
    sc_row = jnp.sum(sc_ref[...], axis=1, keepdims=True)
    row = jnp.where(i < _N_SC_SEG, sc_row, tc_ref[...]) * inv_ref[0]
    msnd_ref[...] = jnp.broadcast_to(row, msnd_ref.shape)


def kernel(Z_img, Z_snd, snd_splits):
    B, C, H, W = Z_img.shape
    N = Z_snd.shape[0]
    n_seg = N // _SEG
    n_tc_seg = n_seg - _N_SC_SEG

    Z_img_2d = Z_img.reshape(B * C * H * W // 128, 128)
    sc_snd, img_sum = _make_sc_kernel(C, B)(Z_snd, Z_img_2d)

    Z_snd_3d = Z_snd.reshape(n_seg, _SEG, C)
    tc_snd = pl.pallas_call(
        _tc_snd_body,
        grid=(n_tc_seg,),
        in_specs=[pl.BlockSpec((1, _SEG, C), lambda i: (i + _N_SC_SEG, 0, 0))],
        out_specs=pl.BlockSpec((1, 1, C), lambda i: (i, 0, 0)),
        out_shape=jax.ShapeDtypeStruct((n_tc_seg, 1, C), jnp.float32),
    )(Z_snd_3d)

    inv = (1.0 / jnp.asarray(snd_splits).astype(jnp.float32)).reshape(1)
    M_img, M_snd = pl.pallas_call(
        _bcast_body,
        grid=(n_seg,),
        in_specs=[
            pl.BlockSpec(memory_space=pltpu.SMEM),
            pl.BlockSpec((B, C), lambda i: (0, 0)),
            pl.BlockSpec((1, _QUARTERS, C),
                         lambda i: (jnp.minimum(i, _N_SC_SEG - 1), 0, 0)),
            pl.BlockSpec((1, 1, C),
                         lambda i: (jnp.maximum(i - _N_SC_SEG, 0), 0, 0)),
        ],
        out_specs=[
            pl.BlockSpec((1, B, C), lambda i: (i, 0, 0)),
            pl.BlockSpec((1, B, C), lambda i: (i, 0, 0)),
        ],
        out_shape=[
            jax.ShapeDtypeStruct((n_seg, B, C), jnp.float32),
            jax.ShapeDtypeStruct((n_seg, B, C), jnp.float32),
        ],
    )(inv, img_sum, sc_snd, tc_snd)
    return (M_img, M_snd)


# speedup vs baseline: 3.1676x; 3.1676x over previous
import functools

import jax
import jax.numpy as jnp
from jax import lax
from jax.experimental import pallas as pl
from jax.experimental.pallas import tpu as pltpu
from jax.experimental.pallas import tpu_sc as plsc

_SEG = 2048
_HW = 196
_SND_CHUNK = 128


def _make_sc_kernel(N, C, n_seg):
    info = plsc.get_sparse_core_info()
    nw = info.num_cores * info.num_subcores
    halves = nw // n_seg
    rows_w = N // nw
    nk = rows_w // _SND_CHUNK
    ng = C // 16
    mesh = plsc.VectorSubcoreMesh(core_axis_name="c", subcore_axis_name="s")

    @functools.partial(
        pl.kernel,
        out_type=jax.ShapeDtypeStruct((n_seg, halves, C), jnp.float32),
        mesh=mesh,
        scratch_types=[
            pltpu.VMEM((2, _SND_CHUNK, C), jnp.float32),
            pltpu.VMEM((C,), jnp.float32),
            pltpu.SemaphoreType.DMA,
            pltpu.SemaphoreType.DMA,
        ],
    )
    def seg_sums(z_hbm, out_hbm, buf, row_v, sem0, sem1):
        wid = lax.axis_index("s") * info.num_cores + lax.axis_index("c")
        base = wid * rows_w
        sems = (sem0, sem1)

        def copy(k):
            return pltpu.make_async_copy(
                z_hbm.at[pl.ds(base + k * _SND_CHUNK, _SND_CHUNK), :],
                buf.at[k % 2], sems[k % 2])

        copy(0).start()
        accs = tuple(jnp.zeros((16,), jnp.float32) for _ in range(ng))
        for k in range(nk):
            if k + 1 < nk:
                copy(k + 1).start()
            copy(k).wait()
            slot = buf.at[k % 2]

            def body(r, a, slot=slot):
                return tuple(
                    a[c] + slot[r, c * 16:(c + 1) * 16] for c in range(ng))

            accs = lax.fori_loop(0, _SND_CHUNK, body, accs)
        for c in range(ng):
            row_v[c * 16:(c + 1) * 16] = accs[c]
        pltpu.sync_copy(row_v, out_hbm.at[wid // halves, wid % halves])

    return seg_sums


def _fused_body(inv_ref, x_ref, snd_ref, mimg_ref, msnd_ref):
    m = jnp.sum(x_ref[...], axis=2) * (1.0 / _HW)
    mimg_ref[...] = jnp.broadcast_to(m[None, :, :], mimg_ref.shape)
    rows = jnp.sum(snd_ref[...], axis=1, keepdims=True) * inv_ref[0]
    msnd_ref[...] = jnp.broadcast_to(rows, msnd_ref.shape)


def kernel(Z_img, Z_snd, snd_splits):
    B, C, H, W = Z_img.shape
    N = Z_snd.shape[0]
    n_seg = N // _SEG

    snd_part = _make_sc_kernel(N, C, n_seg)(Z_snd)

    Z_img_flat = Z_img.reshape(B, C, H * W)
    inv = (1.0 / jnp.asarray(snd_splits).astype(jnp.float32)).reshape(1)
    M_img, M_snd = pl.pallas_call(
        _fused_body,
        grid=(B // 8,),
        in_specs=[
            pl.BlockSpec(memory_space=pltpu.SMEM),
            pl.BlockSpec((8, C, H * W), lambda i: (i, 0, 0)),
            pl.BlockSpec((n_seg, 2, C), lambda i: (0, 0, 0)),
        ],
        out_specs=[
            pl.BlockSpec((n_seg, 8, C), lambda i: (0, i, 0)),
            pl.BlockSpec((n_seg, 8, C), lambda i: (0, i, 0)),
        ],
        out_shape=[
            jax.ShapeDtypeStruct((n_seg, B, C), jnp.float32),
            jax.ShapeDtypeStruct((n_seg, B, C), jnp.float32),
        ],
    )(inv, Z_img_flat, snd_part)
    return (M_img, M_snd)
